# baseline (speedup 1.0000x reference)
import functools

import jax
import jax.numpy as jnp
from jax import lax
from jax.experimental import pallas as pl
from jax.experimental.pallas import tpu as pltpu
from jax.experimental.pallas import tpu_sc as plsc

N = 10000
D_IN = 128
H = 64
C = 64
K = 10
ALPHA = 0.1
EPS = 1e-5

NC = 2
NS = 16
NT = NC * NS
CH = 128
N_PAD = 10240
ROWS_PS = N_PAD // NS


NBUF = 4


def _prop_body(nchunks, t_hbm, src_hbm, dst_hbm, zeros_hbm, out_hbm,
               acc, src_v, dst_v, rows, gsem, ssem):
  c = lax.axis_index("c")
  s = lax.axis_index("s")
  wid = c * NS + s
  pltpu.sync_copy(src_hbm.at[wid], src_v)
  pltpu.sync_copy(dst_hbm.at[wid], dst_v)
  pltpu.sync_copy(zeros_hbm.at[pl.ds(s * ROWS_PS, ROWS_PS)],
                  acc.at[pl.ds(s * ROWS_PS, ROWS_PS)])
  plsc.subcore_barrier()

  def body(j, carry):
    pltpu.async_copy(t_hbm.at[src_v.at[j]], rows.at[0], gsem.at[0]).wait()
    pltpu.sync_copy(rows.at[0], acc.at[dst_v.at[j]], add=True)
    return carry

  lax.fori_loop(0, nchunks, body, 0)
  plsc.subcore_barrier()
  pltpu.sync_copy(acc.at[pl.ds(s * ROWS_PS, ROWS_PS)],
                  out_hbm.at[c, pl.ds(s * ROWS_PS, ROWS_PS)])


@functools.lru_cache(maxsize=None)
def _make_prop(nchunks):
  mesh = plsc.VectorSubcoreMesh(core_axis_name="c", subcore_axis_name="s")
  return pl.kernel(
      functools.partial(_prop_body, nchunks),
      out_type=jax.ShapeDtypeStruct((NC, N_PAD, H), jnp.float32),
      mesh=mesh,
      compiler_params=pltpu.CompilerParams(use_tc_tiling_on_sc=False),
      scratch_types=[
          pltpu.VMEM_SHARED((N_PAD, H), jnp.float32),
          pltpu.VMEM((nchunks, CH), jnp.int32),
          pltpu.VMEM((nchunks, CH), jnp.int32),
          pltpu.VMEM((NBUF, CH, H), jnp.float32),
          pltpu.SemaphoreType.DMA((NBUF,)),
          pltpu.SemaphoreType.DMA((NBUF,)),
      ],
  )


def _prep_body(x_ref, w_ref, degp_ref, t_ref, dinv_ref):
  deg = degp_ref[0, :N] + degp_ref[1, :N]
  dinv = lax.rsqrt(deg)
  dinv_ref[...] = dinv
  t_ref[...] = dinv * jnp.dot(x_ref[...], w_ref[...],
                              preferred_element_type=jnp.float32)


def _layer_mm_body(u_ref, dinv_ref, b_ref, g_ref, be_ref, w_ref,
                   h_ref, t_ref):
  dinv = dinv_ref[...]
  z = dinv * (u_ref[0, :N] + u_ref[1, :N]) + b_ref[...]
  mu = jnp.mean(z, axis=0, keepdims=True)
  var = jnp.mean((z - mu) ** 2, axis=0, keepdims=True)
  h = jnp.maximum((z - mu) * lax.rsqrt(var + EPS) * g_ref[...] + be_ref[...],
                  0.0)
  h_ref[...] = h
  t_ref[...] = dinv * jnp.dot(h, w_ref[...],
                              preferred_element_type=jnp.float32)


def _layer_last_body(u_ref, dinv_ref, b_ref, g_ref, be_ref, h_ref, t_ref):
  dinv = dinv_ref[...]
  z = dinv * (u_ref[0, :N] + u_ref[1, :N]) + b_ref[...]
  mu = jnp.mean(z, axis=0, keepdims=True)
  var = jnp.mean((z - mu) ** 2, axis=0, keepdims=True)
  h = jnp.maximum((z - mu) * lax.rsqrt(var + EPS) * g_ref[...] + be_ref[...],
                  0.0)
  h_ref[...] = h
  t_ref[...] = dinv * h


def _blend_body(u_ref, dinv_ref, h_ref, x_ref, t_ref):
  dinv = dinv_ref[...]
  xn = (1.0 - ALPHA) * (dinv * (u_ref[0, :N] + u_ref[1, :N])) + ALPHA * h_ref[...]
  x_ref[...] = xn
  t_ref[...] = dinv * xn


def _final_body(x_ref, w_ref, b_ref, o_ref):
  h = jnp.dot(x_ref[...], w_ref[...],
              preferred_element_type=jnp.float32) + b_ref[...]
  m = jnp.max(h, axis=1, keepdims=True)
  lse = jnp.log(jnp.sum(jnp.exp(h - m), axis=1, keepdims=True)) + m
  o_ref[...] = h - lse


def _f32(*shape):
  return jax.ShapeDtypeStruct(shape, jnp.float32)


_prep = pl.pallas_call(_prep_body, out_shape=[_f32(N, H), _f32(N, H)])
_layer_mm = pl.pallas_call(_layer_mm_body, out_shape=[_f32(N, H), _f32(N, H)])
_layer_last = pl.pallas_call(_layer_last_body,
                             out_shape=[_f32(N, H), _f32(N, H)])
_blend = pl.pallas_call(_blend_body, out_shape=[_f32(N, H), _f32(N, H)])
_final = pl.pallas_call(_final_body, out_shape=_f32(N, C))


def kernel(x, edge_index, W1, b1, W2, b2, Wc1, bc1, Wc2, bc2,
           g1, be1, g2, be2, g3, be3, Wfc, bfc):
  e_real = edge_index.shape[1] + N
  nchunks = -(-e_real // (NT * CH))
  nchunks = -(-nchunks // NBUF) * NBUF
  e_pad = NT * nchunks * CH
  loop = jnp.arange(N, dtype=jnp.int32)
  pad = e_pad - e_real
  src = jnp.concatenate(
      [edge_index[0], loop, jnp.zeros((pad,), jnp.int32)])
  dst = jnp.concatenate(
      [edge_index[1], loop, jnp.full((pad,), N, jnp.int32)])
  src = src.reshape(NT, nchunks, CH)
  dst = dst.reshape(NT, nchunks, CH)
  zeros_acc = jnp.zeros((N_PAD, H), jnp.float32)
  ones_t = jnp.ones((N, H), jnp.float32)
  prop = _make_prop(nchunks)

  b1r, b2r, bc1r, bc2r = (v.reshape(1, H) for v in (b1, b2, bc1, bc2))
  g1r, be1r, g3r, be3r = (v.reshape(1, H) for v in (g1, be1, g3, be3))
  g2r, be2r = g2.reshape(1, C), be2.reshape(1, C)
  bfcr = bfc.reshape(1, C)

  degp = prop(ones_t, src, dst, zeros_acc)
  t, dinv = _prep(x, W1, degp)
  u = prop(t, src, dst, zeros_acc)
  _, t = _layer_mm(u, dinv, b1r, g1r, be1r, W2)
  u = prop(t, src, dst, zeros_acc)
  _, t = _layer_mm(u, dinv, b2r, g2r, be2r, Wc1)
  u = prop(t, src, dst, zeros_acc)
  _, t = _layer_mm(u, dinv, bc1r, g3r, be3r, Wc2)
  u = prop(t, src, dst, zeros_acc)
  h4, t = _layer_last(u, dinv, bc2r, g3r, be3r)
  xk = h4
  for _ in range(K):
    u = prop(t, src, dst, zeros_acc)
    xk, t = _blend(u, dinv, h4)
  return _final(xk, Wfc, bfcr)


# speedup vs baseline: 1.4279x; 1.4279x over previous
import functools

import jax
import jax.numpy as jnp
from jax import lax
from jax.experimental import pallas as pl
from jax.experimental.pallas import tpu as pltpu
from jax.experimental.pallas import tpu_sc as plsc

N = 10000
D_IN = 128
H = 64
C = 64
K = 10
ALPHA = 0.1
EPS = 1e-5

NC = 2
NS = 16
NT = NC * NS
CH = 128
N_PAD = 10240
ROWS_PS = N_PAD // NS


NBUF = 4


def _prop_body(nchunks, t_hbm, src_hbm, dst_hbm, zeros_hbm, out_hbm,
               acc, idxs, rows, gsem):
  c = lax.axis_index("c")
  s = lax.axis_index("s")
  wid = c * NS + s
  pltpu.sync_copy(zeros_hbm.at[pl.ds(s * ROWS_PS, ROWS_PS)],
                  acc.at[pl.ds(s * ROWS_PS, ROWS_PS)])
  plsc.subcore_barrier()

  def stage(j, b):
    pltpu.sync_copy(src_hbm.at[wid, j], idxs.at[b, 0])
    pltpu.sync_copy(dst_hbm.at[wid, j], idxs.at[b, 1])
    pltpu.async_copy(t_hbm.at[idxs.at[b, 0]], rows.at[b], gsem.at[b])

  stage(0, 0)

  def body(g, carry):
    for b in (0, 1):
      j = 2 * g + b
      pltpu.make_async_copy(t_hbm.at[idxs.at[b, 0]], rows.at[b],
                            gsem.at[b]).wait()

      @pl.when(j + 1 < nchunks)
      def _():
        stage(j + 1, 1 - b)
      pltpu.sync_copy(rows.at[b], acc.at[idxs.at[b, 1]], add=True)
    return carry

  lax.fori_loop(0, nchunks // 2, body, 0)
  plsc.subcore_barrier()
  pltpu.sync_copy(acc.at[pl.ds(s * ROWS_PS, ROWS_PS)],
                  out_hbm.at[c, pl.ds(s * ROWS_PS, ROWS_PS)])


@functools.lru_cache(maxsize=None)
def _make_prop(nchunks):
  mesh = plsc.VectorSubcoreMesh(core_axis_name="c", subcore_axis_name="s")
  return pl.kernel(
      functools.partial(_prop_body, nchunks),
      out_type=jax.ShapeDtypeStruct((NC, N_PAD, H), jnp.float32),
      mesh=mesh,
      compiler_params=pltpu.CompilerParams(use_tc_tiling_on_sc=False),
      scratch_types=[
          pltpu.VMEM_SHARED((N_PAD, H), jnp.float32),
          pltpu.VMEM((2, 2, CH), jnp.int32),
          pltpu.VMEM((2, CH, H), jnp.float32),
          pltpu.SemaphoreType.DMA((2,)),
      ],
  )


def _prep_body(x_ref, w_ref, degp_ref, t_ref, dinv_ref):
  deg = degp_ref[0, :N] + degp_ref[1, :N]
  dinv = lax.rsqrt(deg)
  dinv_ref[...] = dinv
  t_ref[...] = dinv * jnp.dot(x_ref[...], w_ref[...],
                              preferred_element_type=jnp.float32)


def _layer_mm_body(u_ref, dinv_ref, b_ref, g_ref, be_ref, w_ref,
                   h_ref, t_ref):
  dinv = dinv_ref[...]
  z = dinv * (u_ref[0, :N] + u_ref[1, :N]) + b_ref[...]
  mu = jnp.mean(z, axis=0, keepdims=True)
  var = jnp.mean((z - mu) ** 2, axis=0, keepdims=True)
  h = jnp.maximum((z - mu) * lax.rsqrt(var + EPS) * g_ref[...] + be_ref[...],
                  0.0)
  h_ref[...] = h
  t_ref[...] = dinv * jnp.dot(h, w_ref[...],
                              preferred_element_type=jnp.float32)


def _layer_last_body(u_ref, dinv_ref, b_ref, g_ref, be_ref, h_ref, t_ref):
  dinv = dinv_ref[...]
  z = dinv * (u_ref[0, :N] + u_ref[1, :N]) + b_ref[...]
  mu = jnp.mean(z, axis=0, keepdims=True)
  var = jnp.mean((z - mu) ** 2, axis=0, keepdims=True)
  h = jnp.maximum((z - mu) * lax.rsqrt(var + EPS) * g_ref[...] + be_ref[...],
                  0.0)
  h_ref[...] = h
  t_ref[...] = dinv * h


def _blend_body(u_ref, dinv_ref, h_ref, x_ref, t_ref):
  dinv = dinv_ref[...]
  xn = (1.0 - ALPHA) * (dinv * (u_ref[0, :N] + u_ref[1, :N])) + ALPHA * h_ref[...]
  x_ref[...] = xn
  t_ref[...] = dinv * xn


def _final_body(x_ref, w_ref, b_ref, o_ref):
  h = jnp.dot(x_ref[...], w_ref[...],
              preferred_element_type=jnp.float32) + b_ref[...]
  m = jnp.max(h, axis=1, keepdims=True)
  lse = jnp.log(jnp.sum(jnp.exp(h - m), axis=1, keepdims=True)) + m
  o_ref[...] = h - lse


def _f32(*shape):
  return jax.ShapeDtypeStruct(shape, jnp.float32)


_prep = pl.pallas_call(_prep_body, out_shape=[_f32(N, H), _f32(N, H)])
_layer_mm = pl.pallas_call(_layer_mm_body, out_shape=[_f32(N, H), _f32(N, H)])
_layer_last = pl.pallas_call(_layer_last_body,
                             out_shape=[_f32(N, H), _f32(N, H)])
_blend = pl.pallas_call(_blend_body, out_shape=[_f32(N, H), _f32(N, H)])
_final = pl.pallas_call(_final_body, out_shape=_f32(N, C))


def kernel(x, edge_index, W1, b1, W2, b2, Wc1, bc1, Wc2, bc2,
           g1, be1, g2, be2, g3, be3, Wfc, bfc):
  e_real = edge_index.shape[1] + N
  nchunks = -(-e_real // (NT * CH))
  nchunks = -(-nchunks // 2) * 2
  e_pad = NT * nchunks * CH
  loop = jnp.arange(N, dtype=jnp.int32)
  pad = e_pad - e_real
  src = jnp.concatenate(
      [edge_index[0], loop, jnp.zeros((pad,), jnp.int32)])
  dst = jnp.concatenate(
      [edge_index[1], loop, jnp.full((pad,), N, jnp.int32)])
  src = src.reshape(NT, nchunks, CH)
  dst = dst.reshape(NT, nchunks, CH)
  zeros_acc = jnp.zeros((N_PAD, H), jnp.float32)
  ones_t = jnp.ones((N, H), jnp.float32)
  prop = _make_prop(nchunks)

  b1r, b2r, bc1r, bc2r = (v.reshape(1, H) for v in (b1, b2, bc1, bc2))
  g1r, be1r, g3r, be3r = (v.reshape(1, H) for v in (g1, be1, g3, be3))
  g2r, be2r = g2.reshape(1, C), be2.reshape(1, C)
  bfcr = bfc.reshape(1, C)

  degp = prop(ones_t, src, dst, zeros_acc)
  t, dinv = _prep(x, W1, degp)
  u = prop(t, src, dst, zeros_acc)
  _, t = _layer_mm(u, dinv, b1r, g1r, be1r, W2)
  u = prop(t, src, dst, zeros_acc)
  _, t = _layer_mm(u, dinv, b2r, g2r, be2r, Wc1)
  u = prop(t, src, dst, zeros_acc)
  _, t = _layer_mm(u, dinv, bc1r, g3r, be3r, Wc2)
  u = prop(t, src, dst, zeros_acc)
  h4, t = _layer_last(u, dinv, bc2r, g3r, be3r)
  xk = h4
  for _ in range(K):
    u = prop(t, src, dst, zeros_acc)
    xk, t = _blend(u, dinv, h4)
  return _final(xk, Wfc, bfcr)


# speedup vs baseline: 3.1312x; 2.1929x over previous
import functools

import jax
import jax.numpy as jnp
from jax import lax
from jax.experimental import pallas as pl
from jax.experimental.pallas import tpu as pltpu
from jax.experimental.pallas import tpu_sc as plsc

N = 10000
D_IN = 128
H = 64
C = 64
K = 10
ALPHA = 0.1
EPS = 1e-5

NC = 2
NS = 16
NT = NC * NS
CH = 128
N_PAD = 10240
ROWS_PS = N_PAD // NS


BLK = 9


def _prop_body(nblk, t_hbm, src_hbm, dst_hbm, zeros_hbm, out_hbm,
               acc, srcb, dstb, rows, gsem):
  c = lax.axis_index("c")
  s = lax.axis_index("s")
  wid = c * NS + s
  pltpu.sync_copy(zeros_hbm.at[pl.ds(s * ROWS_PS, ROWS_PS)],
                  acc.at[pl.ds(s * ROWS_PS, ROWS_PS)])
  plsc.subcore_barrier()

  def body(k, carry):
    pltpu.sync_copy(src_hbm.at[wid, k], srcb)
    pltpu.sync_copy(dst_hbm.at[wid, k], dstb)
    pltpu.async_copy(t_hbm.at[srcb], rows, gsem).wait()
    pltpu.sync_copy(rows, acc.at[dstb], add=True)
    return carry

  lax.fori_loop(0, nblk, body, 0)
  plsc.subcore_barrier()
  pltpu.sync_copy(acc.at[pl.ds(s * ROWS_PS, ROWS_PS)],
                  out_hbm.at[c, pl.ds(s * ROWS_PS, ROWS_PS)])


@functools.lru_cache(maxsize=None)
def _make_prop(nblk):
  mesh = plsc.VectorSubcoreMesh(core_axis_name="c", subcore_axis_name="s")
  return pl.kernel(
      functools.partial(_prop_body, nblk),
      out_type=jax.ShapeDtypeStruct((NC, N_PAD, H), jnp.float32),
      mesh=mesh,
      compiler_params=pltpu.CompilerParams(use_tc_tiling_on_sc=False),
      scratch_types=[
          pltpu.VMEM_SHARED((N_PAD, H), jnp.float32),
          pltpu.VMEM((BLK * CH,), jnp.int32),
          pltpu.VMEM((BLK * CH,), jnp.int32),
          pltpu.VMEM((BLK * CH, H), jnp.float32),
          pltpu.SemaphoreType.DMA,
      ],
  )


def _prep_body(x_ref, w_ref, degp_ref, t_ref, dinv_ref):
  deg = degp_ref[0, :N] + degp_ref[1, :N]
  dinv = lax.rsqrt(deg)
  dinv_ref[...] = dinv
  t_ref[...] = dinv * jnp.dot(x_ref[...], w_ref[...],
                              preferred_element_type=jnp.float32)


def _layer_mm_body(u_ref, dinv_ref, b_ref, g_ref, be_ref, w_ref,
                   h_ref, t_ref):
  dinv = dinv_ref[...]
  z = dinv * (u_ref[0, :N] + u_ref[1, :N]) + b_ref[...]
  mu = jnp.mean(z, axis=0, keepdims=True)
  var = jnp.mean((z - mu) ** 2, axis=0, keepdims=True)
  h = jnp.maximum((z - mu) * lax.rsqrt(var + EPS) * g_ref[...] + be_ref[...],
                  0.0)
  h_ref[...] = h
  t_ref[...] = dinv * jnp.dot(h, w_ref[...],
                              preferred_element_type=jnp.float32)


def _layer_last_body(u_ref, dinv_ref, b_ref, g_ref, be_ref, h_ref, t_ref):
  dinv = dinv_ref[...]
  z = dinv * (u_ref[0, :N] + u_ref[1, :N]) + b_ref[...]
  mu = jnp.mean(z, axis=0, keepdims=True)
  var = jnp.mean((z - mu) ** 2, axis=0, keepdims=True)
  h = jnp.maximum((z - mu) * lax.rsqrt(var + EPS) * g_ref[...] + be_ref[...],
                  0.0)
  h_ref[...] = h
  t_ref[...] = dinv * h


def _blend_body(u_ref, dinv_ref, h_ref, x_ref, t_ref):
  dinv = dinv_ref[...]
  xn = (1.0 - ALPHA) * (dinv * (u_ref[0, :N] + u_ref[1, :N])) + ALPHA * h_ref[...]
  x_ref[...] = xn
  t_ref[...] = dinv * xn


def _final_body(x_ref, w_ref, b_ref, o_ref):
  h = jnp.dot(x_ref[...], w_ref[...],
              preferred_element_type=jnp.float32) + b_ref[...]
  m = jnp.max(h, axis=1, keepdims=True)
  lse = jnp.log(jnp.sum(jnp.exp(h - m), axis=1, keepdims=True)) + m
  o_ref[...] = h - lse


def _f32(*shape):
  return jax.ShapeDtypeStruct(shape, jnp.float32)


_prep = pl.pallas_call(_prep_body, out_shape=[_f32(N, H), _f32(N, H)])
_layer_mm = pl.pallas_call(_layer_mm_body, out_shape=[_f32(N, H), _f32(N, H)])
_layer_last = pl.pallas_call(_layer_last_body,
                             out_shape=[_f32(N, H), _f32(N, H)])
_blend = pl.pallas_call(_blend_body, out_shape=[_f32(N, H), _f32(N, H)])
_final = pl.pallas_call(_final_body, out_shape=_f32(N, C))


def kernel(x, edge_index, W1, b1, W2, b2, Wc1, bc1, Wc2, bc2,
           g1, be1, g2, be2, g3, be3, Wfc, bfc):
  e_real = edge_index.shape[1] + N
  nblk = -(-e_real // (NT * BLK * CH))
  e_pad = NT * nblk * BLK * CH
  loop = jnp.arange(N, dtype=jnp.int32)
  pad = e_pad - e_real
  src = jnp.concatenate(
      [edge_index[0], loop, jnp.zeros((pad,), jnp.int32)])
  dst = jnp.concatenate(
      [edge_index[1], loop, jnp.full((pad,), N, jnp.int32)])
  src = src.reshape(NT, nblk, BLK * CH)
  dst = dst.reshape(NT, nblk, BLK * CH)
  zeros_acc = jnp.zeros((N_PAD, H), jnp.float32)
  ones_t = jnp.ones((N, H), jnp.float32)
  prop = _make_prop(nblk)

  b1r, b2r, bc1r, bc2r = (v.reshape(1, H) for v in (b1, b2, bc1, bc2))
  g1r, be1r, g3r, be3r = (v.reshape(1, H) for v in (g1, be1, g3, be3))
  g2r, be2r = g2.reshape(1, C), be2.reshape(1, C)
  bfcr = bfc.reshape(1, C)

  degp = prop(ones_t, src, dst, zeros_acc)
  t, dinv = _prep(x, W1, degp)
  u = prop(t, src, dst, zeros_acc)
  _, t = _layer_mm(u, dinv, b1r, g1r, be1r, W2)
  u = prop(t, src, dst, zeros_acc)
  _, t = _layer_mm(u, dinv, b2r, g2r, be2r, Wc1)
  u = prop(t, src, dst, zeros_acc)
  _, t = _layer_mm(u, dinv, bc1r, g3r, be3r, Wc2)
  u = prop(t, src, dst, zeros_acc)
  h4, t = _layer_last(u, dinv, bc2r, g3r, be3r)
  xk = h4
  for _ in range(K):
    u = prop(t, src, dst, zeros_acc)
    xk, t = _blend(u, dinv, h4)
  return _final(xk, Wfc, bfcr)
